# baseline (speedup 1.0000x reference)
import functools

import jax
import jax.numpy as jnp
from jax import lax
from jax.experimental import pallas as pl
from jax.experimental.pallas import tpu as pltpu
from jax.experimental.pallas import tpu_sc as plsc

_BATCH = 16384
_N_FIELDS = 26
_EMBED_DIM = 32
_VOCAB = 100000
_N = _BATCH * _N_FIELDS
_NC = 2
_NS = 16
_NW = _NC * _NS
_PER_W = _N // _NW
_IDX_ROWS = _PER_W // 128
_GROUP_B = 512
_GATHERS = _GROUP_B // 128
_NGROUPS = _PER_W // _GROUP_B
_BTILES = _BATCH // 128
_OUT_ROWS = _N_FIELDS * 4 * _BTILES * 8

_mesh = plsc.VectorSubcoreMesh(core_axis_name="c", subcore_axis_name="s")


@functools.partial(
    pl.kernel,
    out_type=jax.ShapeDtypeStruct((_OUT_ROWS, 128), jnp.float32),
    mesh=_mesh,
    scratch_types=[
        pltpu.VMEM((_IDX_ROWS, 128), jnp.int32),
        pltpu.VMEM((_GROUP_B, _EMBED_DIM), jnp.float32),
        pltpu.VMEM((_GROUP_B, _EMBED_DIM), jnp.float32),
        pltpu.VMEM((128, 128), jnp.float32),
        pltpu.VMEM((128, 128), jnp.float32),
        pltpu.SemaphoreType.DMA,
        pltpu.SemaphoreType.DMA,
        pltpu.SemaphoreType.DMA,
        pltpu.SemaphoreType.DMA,
    ],
    compiler_params=pltpu.CompilerParams(
        use_tc_tiling_on_sc=False, needs_layout_passes=False
    ),
)
def _embed_kernel(x2_hbm, table_hbm, out_hbm, idx_v, rows_a, rows_b,
                  out_a, out_b, gsem_a, gsem_b, wsem_a, wsem_b):
    wid = lax.axis_index("s") * _NC + lax.axis_index("c")
    row0 = wid * _IDX_ROWS

    pltpu.sync_copy(x2_hbm.at[pl.ds(row0, _IDX_ROWS)], idx_v)

    iota = lax.iota(jnp.int32, 16)

    def _adjust(r, carry):
        off = ((row0 + r) // _BTILES) * _VOCAB
        offv = jnp.full((16,), off, jnp.int32)
        for c in range(8):
            sl = pl.ds(c * 16, 16)
            idx_v[r, sl] = idx_v[r, sl] + offv
        return carry

    lax.fori_loop(0, _IDX_ROWS, _adjust, 0)

    def _fire_gathers(k, rows, gsem):
        for q in range(_GATHERS):
            pltpu.async_copy(
                table_hbm.at[idx_v.at[k * _GATHERS + q]],
                rows.at[pl.ds(q * 128, 128)],
                gsem,
            )

    def _wait_gathers(rows, gsem):
        pltpu.make_async_copy(
            table_hbm.at[pl.ds(0, _GROUP_B)], rows, gsem
        ).wait()

    def _transpose(rows, out_t):
        def _body(i, carry):
            bh = i // 8
            j = i % 8
            b_idx = iota + i * 16
            for e in range(_EMBED_DIM):
                e_vec = jnp.full((16,), e, jnp.int32)
                v = plsc.load_gather(rows, [b_idx, e_vec])
                r_out = (e // 8) * 32 + bh * 8 + (e % 8)
                out_t[r_out, pl.ds(j * 16, 16)] = v
            return carry

        lax.fori_loop(0, 32, _body, 0)

    def _fire_writes(k, out_t, wsem):
        gid = (row0 + k * _GATHERS) // _BTILES
        f = gid
        bt0 = (row0 + k * _GATHERS) - f * _BTILES
        for e_hi in range(4):
            rbase = ((f * 4 + e_hi) * _BTILES + bt0) * 8
            pltpu.async_copy(
                out_t.at[pl.ds(e_hi * 32, 32)],
                out_hbm.at[pl.ds(rbase, 32)],
                wsem,
            )

    def _wait_writes(out_t, wsem):
        pltpu.make_async_copy(
            out_hbm.at[pl.ds(0, 128)], out_t, wsem
        ).wait()

    _fire_gathers(0, rows_a, gsem_a)

    def _pair(p, carry):
        ka = 2 * p
        kb = 2 * p + 1

        @pl.when(p > 0)
        def _():
            _wait_writes(out_b, wsem_b)

        _fire_gathers(kb, rows_b, gsem_b)
        _wait_gathers(rows_a, gsem_a)

        @pl.when(p > 0)
        def _():
            _wait_writes(out_a, wsem_a)

        _transpose(rows_a, out_a)
        _fire_writes(ka, out_a, wsem_a)

        @pl.when(p < _NGROUPS // 2 - 1)
        def _():
            _fire_gathers(ka + 2, rows_a, gsem_a)

        _wait_gathers(rows_b, gsem_b)
        _transpose(rows_b, out_b)
        _fire_writes(kb, out_b, wsem_b)
        return carry

    lax.fori_loop(0, _NGROUPS // 2, _pair, 0)
    _wait_writes(out_a, wsem_a)
    _wait_writes(out_b, wsem_b)


def kernel(x, embedding_table):
    x2 = x.T.reshape(_N // 128, 128)
    out2 = _embed_kernel(x2, embedding_table)
    out5 = out2.reshape(_N_FIELDS, 4, _BTILES, 8, 128)
    return out5.transpose(2, 4, 0, 1, 3).reshape(_BATCH, _N_FIELDS, _EMBED_DIM)


# speedup vs baseline: 1.0181x; 1.0181x over previous
import functools

import jax
import jax.numpy as jnp
import numpy as np
from jax import lax
from jax.experimental import pallas as pl
from jax.experimental.pallas import tpu as pltpu
from jax.experimental.pallas import tpu_sc as plsc

_BATCH = 16384
_N_FIELDS = 26
_EMBED_DIM = 32
_VOCAB = 100000
_N = _BATCH * _N_FIELDS
_NC = 2
_NS = 16
_NW = _NC * _NS
_PER_W = _N // _NW
_IDX_ROWS = _PER_W // 128
_B_PER_W = _BATCH // _NW
_CHUNK = 1024
_GATHERS = _CHUNK // 128
_NCHUNKS = _PER_W // _CHUNK

_p = np.arange(_PER_W, dtype=np.int64)
_lb = _p // _N_FIELDS
_f = _p % _N_FIELDS
_SRC = ((_f // 8) * 4096 + (_lb // 128) * 1024 + (_f % 8) * 128
        + (_lb % 128)).astype(np.int32).reshape(_IDX_ROWS, 128)
_OFF = (_f * _VOCAB).astype(np.int32).reshape(_IDX_ROWS, 128)

_mesh = plsc.VectorSubcoreMesh(core_axis_name="c", subcore_axis_name="s")


@functools.partial(
    pl.kernel,
    out_type=jax.ShapeDtypeStruct((_N, _EMBED_DIM), jnp.float32),
    mesh=_mesh,
    scratch_types=[
        pltpu.VMEM((128, 128), jnp.int32),
        pltpu.VMEM((_IDX_ROWS, 128), jnp.int32),
        pltpu.VMEM((_IDX_ROWS, 128), jnp.int32),
        pltpu.VMEM((_IDX_ROWS, 128), jnp.int32),
        pltpu.VMEM((_CHUNK, _EMBED_DIM), jnp.float32),
        pltpu.SemaphoreType.DMA,
        pltpu.SemaphoreType.DMA,
    ],
    compiler_params=pltpu.CompilerParams(
        use_tc_tiling_on_sc=False, needs_layout_passes=False
    ),
)
def _embed_kernel(x4_hbm, src_hbm, off_hbm, table_hbm, out_hbm,
                  x_raw, m_v, off_v, idx_v, buf_v, ssem, gsem):
    wid = lax.axis_index("s") * _NC + lax.axis_index("c")
    base = wid * _PER_W

    for f_hi in range(4):
        for bh in range(4):
            r0 = (f_hi * 128 + wid * 4 + bh) * 8
            pltpu.async_copy(
                x4_hbm.at[pl.ds(r0, 8)],
                x_raw.at[pl.ds(f_hi * 32 + bh * 8, 8)],
                ssem,
            )
    pltpu.sync_copy(src_hbm, m_v)
    pltpu.sync_copy(off_hbm, off_v)
    pltpu.make_async_copy(x4_hbm.at[pl.ds(0, 128)], x_raw, ssem).wait()

    def _build(r, carry):
        for c in range(8):
            sl = pl.ds(c * 16, 16)
            m = m_v[r, sl]
            v = plsc.load_gather(x_raw, [m >> 7, m & 127])
            idx_v[r, sl] = v + off_v[r, sl]
        return carry

    lax.fori_loop(0, _IDX_ROWS, _build, 0)

    def _chunk(ci, carry):
        for j in range(_GATHERS):
            pltpu.async_copy(
                table_hbm.at[idx_v.at[ci * _GATHERS + j]],
                buf_v.at[pl.ds(j * 128, 128)],
                gsem,
            )
        pltpu.make_async_copy(
            table_hbm.at[pl.ds(0, _CHUNK)], buf_v, gsem
        ).wait()
        pltpu.sync_copy(buf_v, out_hbm.at[pl.ds(base + ci * _CHUNK, _CHUNK)])
        return carry

    lax.fori_loop(0, _NCHUNKS, _chunk, 0)


def kernel(x, embedding_table):
    y = jnp.pad(x.T, ((0, 32 - _N_FIELDS), (0, 0)))
    x4 = y.reshape(4, 8, 128, 128).transpose(0, 2, 1, 3).reshape(4096, 128)
    out = _embed_kernel(
        x4, jnp.asarray(_SRC), jnp.asarray(_OFF), embedding_table
    )
    return out.reshape(_BATCH, _N_FIELDS, _EMBED_DIM)
